# baseline (speedup 1.0000x reference)
import functools

import jax
import jax.numpy as jnp
from jax.experimental import pallas as pl
from jax.experimental.pallas import tpu as pltpu


def _elu(v):
    return jnp.where(v > 0, v, jnp.exp(jnp.minimum(v, 0.0)) - 1.0)


def _make_fused_kernel(block_m):
    def fused_kernel(adj_ref, x_ref, w1_ref, b1_ref, w2_ref, b2_ref, o_ref,
                     sa_ref, sb_ref):
        layer = pl.program_id(0)
        i = pl.program_id(1)

        @pl.when((layer == 0) & (i == 0))
        def _():
            sa_ref[...] = jnp.dot(x_ref[...], w1_ref[...],
                                  preferred_element_type=jnp.float32)

        @pl.when(layer == 0)
        def _():
            acc = jnp.dot(adj_ref[...], sa_ref[...],
                          preferred_element_type=jnp.float32)
            sb_ref[pl.ds(i * block_m, block_m), :] = _elu(acc + b1_ref[...])

        @pl.when((layer == 1) & (i == 0))
        def _():
            sa_ref[...] = jnp.dot(sb_ref[...], w2_ref[...],
                                  preferred_element_type=jnp.float32)

        @pl.when(layer == 1)
        def _():
            acc = jnp.dot(adj_ref[...], sa_ref[...],
                          preferred_element_type=jnp.float32)
            o_ref[...] = _elu(acc + b2_ref[...])

    return fused_kernel


@functools.partial(jax.jit, static_argnames=("block_m",))
def _forward(x, adj, W1, b1, W2, b2, block_m=400):
    n, _ = x.shape
    nhid = W1.shape[1]

    return pl.pallas_call(
        _make_fused_kernel(block_m),
        grid=(2, n // block_m),
        in_specs=[
            pl.BlockSpec((block_m, n), lambda l, i: (i, 0)),
            pl.BlockSpec(x.shape, lambda l, i: (0, 0)),
            pl.BlockSpec(W1.shape, lambda l, i: (0, 0)),
            pl.BlockSpec((1, nhid), lambda l, i: (0, 0)),
            pl.BlockSpec(W2.shape, lambda l, i: (0, 0)),
            pl.BlockSpec((1, nhid), lambda l, i: (0, 0)),
        ],
        out_specs=pl.BlockSpec((block_m, nhid), lambda l, i: (l * i, 0)),
        out_shape=jax.ShapeDtypeStruct((n, nhid), jnp.float32),
        scratch_shapes=[
            pltpu.VMEM((n, nhid), jnp.float32),
            pltpu.VMEM((n, nhid), jnp.float32),
        ],
    )(adj, x, W1, b1.reshape(1, nhid), W2, b2.reshape(1, nhid))


def kernel(x, adj, W1, b1, W2, b2):
    return _forward(x, adj, W1, b1, W2, b2)
